# baseline (speedup 1.0000x reference)
import jax
import jax.numpy as jnp
from jax.experimental import pallas as pl
from jax.experimental.pallas import tpu as pltpu

B, H, S, D = 8, 16, 2048, 128
L = 16
BH = B * H
RB = 2


def _body(pos_ref, kc, vc, kv, vv, ko, vo):
    ko[...] = kc[...]
    vo[...] = vc[...]
    for rb in range(RB):
        for i in range(L):
            r = pos_ref[i]
            ko[rb, pl.ds(r, 1), :] = kv[rb, pl.ds(i, 1), :]
            vo[rb, pl.ds(r, 1), :] = vv[rb, pl.ds(i, 1), :]


@jax.jit
def _run(input_pos, k_val, v_val, k_cache, v_cache):
    kc = k_cache.reshape(BH, S, D)
    vc = v_cache.reshape(BH, S, D)
    kv = k_val.reshape(BH, L, D)
    vv = v_val.reshape(BH, L, D)

    grid_spec = pltpu.PrefetchScalarGridSpec(
        num_scalar_prefetch=1,
        grid=(BH // RB,),
        in_specs=[
            pl.BlockSpec((RB, S, D), lambda i, pos: (i, 0, 0)),
            pl.BlockSpec((RB, S, D), lambda i, pos: (i, 0, 0)),
            pl.BlockSpec((RB, L, D), lambda i, pos: (i, 0, 0)),
            pl.BlockSpec((RB, L, D), lambda i, pos: (i, 0, 0)),
        ],
        out_specs=[
            pl.BlockSpec((RB, S, D), lambda i, pos: (i, 0, 0)),
            pl.BlockSpec((RB, S, D), lambda i, pos: (i, 0, 0)),
        ],
    )
    ko, vo = pl.pallas_call(
        _body,
        grid_spec=grid_spec,
        out_shape=[
            jax.ShapeDtypeStruct((BH, S, D), jnp.float32),
            jax.ShapeDtypeStruct((BH, S, D), jnp.float32),
        ],
    )(input_pos, kc, vc, kv, vv)
    return ko.reshape(B, H, S, D), vo.reshape(B, H, S, D)


def kernel(input_pos, k_val, v_val, k_cache, v_cache):
    return _run(input_pos, k_val, v_val, k_cache, v_cache)


# speedup vs baseline: 1.0150x; 1.0150x over previous
import jax
import jax.numpy as jnp
from jax.experimental import pallas as pl
from jax.experimental.pallas import tpu as pltpu

B, H, S, D = 8, 16, 2048, 128
L = 16
BH = B * H
RB = 4


def _body(pos_ref, kc, vc, kv, vv, ko, vo):
    ko[...] = kc[...]
    vo[...] = vc[...]
    for rb in range(RB):
        for i in range(L):
            r = pos_ref[i]
            ko[rb, pl.ds(r, 1), :] = kv[rb, pl.ds(i, 1), :]
            vo[rb, pl.ds(r, 1), :] = vv[rb, pl.ds(i, 1), :]


@jax.jit
def _run(input_pos, k_val, v_val, k_cache, v_cache):
    kc = k_cache.reshape(BH, S, D)
    vc = v_cache.reshape(BH, S, D)
    kv = k_val.reshape(BH, L, D)
    vv = v_val.reshape(BH, L, D)

    grid_spec = pltpu.PrefetchScalarGridSpec(
        num_scalar_prefetch=1,
        grid=(BH // RB,),
        in_specs=[
            pl.BlockSpec((RB, S, D), lambda i, pos: (i, 0, 0)),
            pl.BlockSpec((RB, S, D), lambda i, pos: (i, 0, 0)),
            pl.BlockSpec((RB, L, D), lambda i, pos: (i, 0, 0)),
            pl.BlockSpec((RB, L, D), lambda i, pos: (i, 0, 0)),
        ],
        out_specs=[
            pl.BlockSpec((RB, S, D), lambda i, pos: (i, 0, 0)),
            pl.BlockSpec((RB, S, D), lambda i, pos: (i, 0, 0)),
        ],
    )
    ko, vo = pl.pallas_call(
        _body,
        grid_spec=grid_spec,
        out_shape=[
            jax.ShapeDtypeStruct((BH, S, D), jnp.float32),
            jax.ShapeDtypeStruct((BH, S, D), jnp.float32),
        ],
    )(input_pos, kc, vc, kv, vv)
    return ko.reshape(B, H, S, D), vo.reshape(B, H, S, D)


def kernel(input_pos, k_val, v_val, k_cache, v_cache):
    return _run(input_pos, k_val, v_val, k_cache, v_cache)
